# baseline (speedup 1.0000x reference)
import functools
import math

import jax
import jax.numpy as jnp
from jax import lax
from jax.experimental import pallas as pl
from jax.experimental.pallas import tpu as pltpu
from jax.experimental.pallas import tpu_sc as plsc


def _proj_kernel(x_ref, wqk_ref, bqk_ref, wv_ref, bv_ref, qk_ref, v_ref):
    x = x_ref[...]
    qk_ref[...] = jax.lax.dot_general(
        x, wqk_ref[...], (((1,), (1,)), ((), ())),
        preferred_element_type=jnp.float32) + bqk_ref[...]
    v_ref[...] = jax.lax.dot_general(
        x, wv_ref[...], (((1,), (1,)), ((), ())),
        preferred_element_type=jnp.float32) + bv_ref[...]


def _project(x2, Wqk, bqk, Wv, bv, row_block):
    S, DM = x2.shape
    return pl.pallas_call(
        _proj_kernel,
        grid=(S // row_block,),
        in_specs=[
            pl.BlockSpec((row_block, DM), lambda i: (i, 0)),
            pl.BlockSpec((DM, DM), lambda i: (0, 0)),
            pl.BlockSpec((1, DM), lambda i: (0, 0)),
            pl.BlockSpec((DM, DM), lambda i: (0, 0)),
            pl.BlockSpec((1, DM), lambda i: (0, 0)),
        ],
        out_specs=[
            pl.BlockSpec((row_block, DM), lambda i: (i, 0)),
            pl.BlockSpec((row_block, DM), lambda i: (i, 0)),
        ],
        out_shape=[
            jax.ShapeDtypeStruct((S, DM), jnp.float32),
            jax.ShapeDtypeStruct((S, DM), jnp.float32),
        ],
    )(x2, Wqk, bqk, Wv, bv)


def _hash_kernel(qk_ref, rot_ref, bkt_ref, offs_ref, counts_sc,
                 *, n_hashes, rot_size, n_heads, t):
    hh = pl.program_id(0)
    nb = 2 * rot_size
    q = qk_ref[...]
    r = rot_ref[0]
    rot = jax.lax.dot_general(
        q, r, (((1,), (0,)), ((), ())), preferred_element_type=jnp.float32)
    s = rot.shape[0]

    @pl.when(hh == 0)
    def _():
        counts_sc[...] = jnp.zeros_like(counts_sc)

    lanes = lax.broadcasted_iota(jnp.int32, (s, nb), 1)
    for h in range(n_hashes):
        sub = rot[:, h * rot_size:(h + 1) * rot_size]
        full = jnp.concatenate([sub, -sub], axis=1)
        b = jnp.argmax(full, axis=1).astype(jnp.int32)
        bkt_ref[0, h, :] = b
        oh = (b[:, None] == lanes).astype(jnp.float32)
        counts_sc[h, :] = counts_sc[h, :] + jnp.sum(oh, axis=0)

    @pl.when(hh == n_heads - 1)
    def _():
        cnt = counts_sc[...]
        inc = cnt
        shift = 1
        while shift < nb:
            inc = inc + jnp.concatenate(
                [jnp.zeros((n_hashes, shift), jnp.float32),
                 inc[:, :-shift]], axis=1)
            shift *= 2
        offs = jnp.concatenate(
            [jnp.zeros((n_hashes, 1), jnp.float32), inc[:, :-1]], axis=1)
        pad = jnp.full((n_hashes, nb), float(t), dtype=jnp.float32)
        offs_ref[...] = jnp.concatenate(
            [offs, pad], axis=1).astype(jnp.int32)[:, None, :]


def _hash_buckets(qk_heads, rot_flat, n_hashes, rot_size):
    H = rot_flat.shape[0]
    D = rot_flat.shape[1]
    S = qk_heads.shape[0] // H
    nb = 2 * rot_size
    return pl.pallas_call(
        functools.partial(_hash_kernel, n_hashes=n_hashes, rot_size=rot_size,
                          n_heads=H, t=H * S),
        grid=(H,),
        in_specs=[
            pl.BlockSpec((S, D), lambda h: (h, 0)),
            pl.BlockSpec((1, D, n_hashes * rot_size), lambda h: (h, 0, 0)),
        ],
        out_specs=[
            pl.BlockSpec((1, n_hashes, S), lambda h: (h, 0, 0)),
            pl.BlockSpec((n_hashes, 1, 2 * nb), lambda h: (0, 0, 0)),
        ],
        out_shape=[
            jax.ShapeDtypeStruct((H, n_hashes, S), jnp.int32),
            jax.ShapeDtypeStruct((n_hashes, 1, 2 * nb), jnp.int32),
        ],
        scratch_shapes=[pltpu.VMEM((n_hashes, nb), jnp.float32)],
    )(qk_heads, rot_flat)


def _rank_kernel(tri_ref, bkt_ref, offs_ref, rank_ref, counts_sc,
                 *, cs, nb, t):
    h = pl.program_id(0)
    c = pl.program_id(1)

    b = bkt_ref[0, 0, :]
    lanes = lax.broadcasted_iota(jnp.int32, (cs, nb), 1)
    oh = (b[:, None] == lanes).astype(jnp.float32)

    @pl.when(c == 0)
    def _():
        counts_sc[...] = jnp.zeros_like(counts_sc)

    csum = jax.lax.dot_general(
        tri_ref[...], oh.astype(jnp.bfloat16), (((1,), (0,)), ((), ())),
        preferred_element_type=jnp.float32)
    offs = offs_ref[0, 0, :nb].astype(jnp.float32)[None, :]
    inc_global = csum + counts_sc[...]
    rank_f = jnp.sum(oh * (offs + inc_global - 1.0), axis=1)
    rank_ref[0, 0, :] = (rank_f + 0.5).astype(jnp.int32) + h * t
    counts_sc[...] = counts_sc[...] + jnp.sum(oh, axis=0, keepdims=True)


def _ranks(buckets3, offs3, nb, cs):
    n_hashes, _, T = buckets3.shape
    nc = T // cs
    rr = lax.broadcasted_iota(jnp.int32, (cs, cs), 0)
    cc = lax.broadcasted_iota(jnp.int32, (cs, cs), 1)
    tri = (rr >= cc).astype(jnp.bfloat16)
    return pl.pallas_call(
        functools.partial(_rank_kernel, cs=cs, nb=nb, t=T),
        grid=(n_hashes, nc),
        in_specs=[
            pl.BlockSpec((cs, cs), lambda h, c: (0, 0)),
            pl.BlockSpec((1, 1, cs), lambda h, c: (h, 0, c)),
            pl.BlockSpec((1, 1, 2 * nb), lambda h, c: (h, 0, 0)),
        ],
        out_specs=pl.BlockSpec((1, 1, cs), lambda h, c: (h, 0, c)),
        out_shape=jax.ShapeDtypeStruct((n_hashes, 1, T), jnp.int32),
        scratch_shapes=[pltpu.VMEM((1, nb), jnp.float32)],
        compiler_params=pltpu.CompilerParams(
            dimension_semantics=("parallel", "arbitrary")),
    )(tri, buckets3, offs3)


def _sc_sort_scatter(kv_flat, rank_g):
    T, DK = kv_flat.shape
    NH = rank_g.shape[0]
    info = plsc.get_sparse_core_info()
    nw = info.num_cores * info.num_subcores
    rpw = T // nw
    nch = rpw // 128
    mesh = plsc.VectorSubcoreMesh(core_axis_name="c", subcore_axis_name="s")

    @functools.partial(
        pl.kernel, mesh=mesh,
        out_type=jax.ShapeDtypeStruct((NH * T, DK), jnp.float32),
        scratch_types=[pltpu.VMEM((nch, 128), jnp.int32),
                       pltpu.VMEM((rpw, DK), jnp.float32),
                       pltpu.SemaphoreType.DMA,
                       pltpu.SemaphoreType.DMA],
    )
    def sortk(kv_hbm, rank_hbm, kvs_hbm, idx_v, rows, semi, semw):
        wid = lax.axis_index("s") * info.num_cores + lax.axis_index("c")
        base = wid * rpw
        pltpu.sync_copy(kv_hbm.at[pl.ds(base, rpw)], rows)

        def per_hash(h, carry):
            loads = [
                pltpu.async_copy(rank_hbm.at[h, pl.ds(base + j * 128, 128)],
                                 idx_v.at[j], semi)
                for j in range(nch)
            ]
            for hd in loads:
                hd.wait()
            stores = [
                pltpu.async_copy(rows.at[pl.ds(j * 128, 128)],
                                 kvs_hbm.at[idx_v.at[j]], semw)
                for j in range(nch)
            ]
            for hd in stores:
                hd.wait()
            return carry

        lax.fori_loop(0, NH, per_hash, 0)

    return sortk(kv_flat, rank_g)


def _sc_unsort_gather(out_sorted, rank_g):
    TT, D = out_sorted.shape
    NH, T = rank_g.shape
    info = plsc.get_sparse_core_info()
    nw = info.num_cores * info.num_subcores
    rpw = T // nw
    nch = rpw // 128
    mesh = plsc.VectorSubcoreMesh(core_axis_name="c", subcore_axis_name="s")

    @functools.partial(
        pl.kernel, mesh=mesh,
        out_type=jax.ShapeDtypeStruct((NH * T, D), jnp.float32),
        scratch_types=[pltpu.VMEM((nch, 128), jnp.int32),
                       pltpu.VMEM((rpw, D), jnp.float32),
                       pltpu.SemaphoreType.DMA,
                       pltpu.SemaphoreType.DMA],
    )
    def gatherk(os_hbm, rank_hbm, og_hbm, idx_v, rows, semi, semr):
        wid = lax.axis_index("s") * info.num_cores + lax.axis_index("c")
        base = wid * rpw

        def per_hash(h, carry):
            loads = [
                pltpu.async_copy(rank_hbm.at[h, pl.ds(base + j * 128, 128)],
                                 idx_v.at[j], semi)
                for j in range(nch)
            ]
            for hd in loads:
                hd.wait()
            reads = [
                pltpu.async_copy(os_hbm.at[idx_v.at[j]],
                                 rows.at[pl.ds(j * 128, 128)], semr)
                for j in range(nch)
            ]
            for hd in reads:
                hd.wait()
            pltpu.sync_copy(rows, og_hbm.at[pl.ds(h * T + base, rpw)])
            return carry

        lax.fori_loop(0, NH, per_hash, 0)

    return gatherk(out_sorted, rank_g)


def _prep_kernel(offs_ref, kv_ref, kbf_ref, vaug_ref, bks_ref, mx_ref,
                 maxn_sc, *, rb, nb, d, nc):
    c = pl.program_id(1)
    kv = kv_ref[0]
    kbf_ref[0] = kv[:, :d].astype(jnp.bfloat16)
    lane = lax.broadcasted_iota(jnp.int32, (rb, 2 * d), 1)
    vaug_ref[0] = jnp.where(
        lane < d, jnp.roll(kv, -d, axis=1),
        jnp.where(lane == d, 1.0, 0.0)).astype(jnp.bfloat16)
    off32 = offs_ref[0, 0, :nb]
    p = c * rb + lax.broadcasted_iota(jnp.int32, (rb, 1), 0)
    bks_ref[0, 0, :] = jnp.sum(
        (off32[None, :] <= p).astype(jnp.int32), axis=1) - 1
    k = kv[:, :d]
    n2 = jnp.max(jnp.sum(k * k, axis=1))

    @pl.when(c == 0)
    def _():
        maxn_sc[...] = jnp.zeros_like(maxn_sc)

    maxn_sc[...] = jnp.maximum(maxn_sc[...], n2.reshape(1, 1))

    @pl.when(c == nc - 1)
    def _():
        mx_ref[...] = jnp.broadcast_to(maxn_sc[...], mx_ref.shape)


def _prep(kvs3, offs3, rb):
    n_hashes, T, DK = kvs3.shape
    d = DK // 2
    nb = offs3.shape[2] // 2
    nc = T // rb
    return pl.pallas_call(
        functools.partial(_prep_kernel, rb=rb, nb=nb, d=d, nc=nc),
        grid=(n_hashes, nc),
        in_specs=[
            pl.BlockSpec((1, 1, 2 * nb), lambda h, c: (h, 0, 0)),
            pl.BlockSpec((1, rb, DK), lambda h, c: (h, c, 0)),
        ],
        out_specs=[
            pl.BlockSpec((1, rb, d), lambda h, c: (h, c, 0)),
            pl.BlockSpec((1, rb, DK), lambda h, c: (h, c, 0)),
            pl.BlockSpec((1, 1, rb), lambda h, c: (h, 0, c)),
            pl.BlockSpec((1, 1, 128), lambda h, c: (h, 0, 0)),
        ],
        out_shape=[
            jax.ShapeDtypeStruct((n_hashes, T, d), jnp.bfloat16),
            jax.ShapeDtypeStruct((n_hashes, T, DK), jnp.bfloat16),
            jax.ShapeDtypeStruct((n_hashes, 1, T), jnp.int32),
            jax.ShapeDtypeStruct((n_hashes, 1, 128), jnp.float32),
        ],
        scratch_shapes=[pltpu.VMEM((1, 1), jnp.float32)],
        compiler_params=pltpu.CompilerParams(
            dimension_semantics=("parallel", "arbitrary")),
    )(offs3, kvs3)


def _attn_kernel(offs_ref, bq_ref, mx_ref, q_ref, k_ref, vaug_ref, bks_ref,
                 o_ref, *, bq, bk, nb, t, d, scale):
    qi = pl.program_id(1)
    off = offs_ref[0, 0, :]
    off32 = off[:nb]
    qlo = qi * bq
    qhi = qlo + bq - 1

    kv_start = jnp.max(jnp.where(off32 <= qlo, off32, 0))
    kv_end = jnp.min(jnp.where(off > qhi, off, t))
    ks_blk = kv_start // bk
    ke_blk = (kv_end + bk - 1) // bk

    bq_id = bq_ref[0, 0, :]
    mshift = mx_ref[0, 0, 0] * scale
    q = q_ref[0] * jnp.bfloat16(scale)

    def body(ki, acc):
        koff = ki * bk
        k = k_ref[0, pl.ds(koff, bk), :]
        vaug = vaug_ref[0, pl.ds(koff, bk), :]
        bk_id = bks_ref[0, 0, pl.ds(koff, bk)]
        s = jax.lax.dot_general(
            q, k, (((1,), (1,)), ((), ())),
            preferred_element_type=jnp.float32)
        mask = bq_id[:, None] == bk_id[None, :]
        p = jnp.exp(jnp.where(mask, s - mshift, -1e30))
        return acc + jax.lax.dot_general(
            p.astype(jnp.bfloat16), vaug, (((1,), (0,)), ((), ())),
            preferred_element_type=jnp.float32)

    a0 = jnp.zeros((bq, 2 * d), dtype=jnp.float32)
    acc = lax.fori_loop(ks_blk, ke_blk, body, a0)
    o_ref[0, :, :d] = acc[:, :d] / acc[:, d:d + 1]


def _attention_sorted(kbf3, vaug3, bks3, offs3, mx3, bq_block, bk_block):
    n_hashes, T, d = kbf3.shape
    DK = 2 * d
    nb = offs3.shape[2] // 2
    nq = T // bq_block
    scale = 1.0 / math.sqrt(d)
    return pl.pallas_call(
        functools.partial(_attn_kernel, bq=bq_block, bk=bk_block,
                          nb=nb, t=T, d=d, scale=scale),
        grid=(n_hashes, nq),
        in_specs=[
            pl.BlockSpec((1, 1, 2 * nb), lambda h, qi: (h, 0, 0)),
            pl.BlockSpec((1, 1, bq_block), lambda h, qi: (h, 0, qi)),
            pl.BlockSpec((1, 1, 128), lambda h, qi: (h, 0, 0)),
            pl.BlockSpec((1, bq_block, d), lambda h, qi: (h, qi, 0)),
            pl.BlockSpec((1, T, d), lambda h, qi: (h, 0, 0)),
            pl.BlockSpec((1, T, DK), lambda h, qi: (h, 0, 0)),
            pl.BlockSpec((1, 1, T), lambda h, qi: (h, 0, 0)),
        ],
        out_specs=pl.BlockSpec((1, bq_block, DK), lambda h, qi: (h, qi, 0)),
        out_shape=jax.ShapeDtypeStruct((n_hashes, T, DK), jnp.float32),
        compiler_params=pltpu.CompilerParams(
            dimension_semantics=("parallel", "parallel")),
    )(offs3, bks3, mx3, kbf3, kbf3, vaug3, bks3)


def _sum_kernel(g_ref, o_ref, *, inv_nh, d):
    o_ref[...] = jnp.sum(g_ref[...][:, :, :d], axis=0) * inv_nh


def _sum_hashes(og3, row_block):
    NH, T, DK = og3.shape
    d = DK // 2
    return pl.pallas_call(
        functools.partial(_sum_kernel, inv_nh=1.0 / NH, d=d),
        grid=(T // row_block,),
        in_specs=[pl.BlockSpec((NH, row_block, DK), lambda i: (0, i, 0))],
        out_specs=pl.BlockSpec((row_block, d), lambda i: (i, 0)),
        out_shape=jax.ShapeDtypeStruct((T, d), jnp.float32),
    )(og3)


def _outproj_kernel(y_ref, wo_ref, bo_ref, o_ref):
    o_ref[...] = jax.lax.dot_general(
        y_ref[...], wo_ref[...], (((1,), (1,)), ((), ())),
        preferred_element_type=jnp.float32) + bo_ref[...]


def _outproj(y2, Wo, bo, row_block):
    S, DM = y2.shape
    return pl.pallas_call(
        _outproj_kernel,
        grid=(S // row_block,),
        in_specs=[
            pl.BlockSpec((row_block, DM), lambda i: (i, 0)),
            pl.BlockSpec((DM, DM), lambda i: (0, 0)),
            pl.BlockSpec((1, DM), lambda i: (0, 0)),
        ],
        out_specs=pl.BlockSpec((row_block, DM), lambda i: (i, 0)),
        out_shape=jax.ShapeDtypeStruct((S, DM), jnp.float32),
    )(y2, Wo, bo)


def kernel(x, Wqk, bqk, Wv, bv, Wo, bo, rotations):
    batch, S, DM = x.shape
    n_hashes, H, D, C = rotations.shape
    T = batch * H * S
    nb = 2 * C

    x2 = x.reshape(batch * S, DM)
    row_block = min(256, batch * S)
    qk2, v2 = _project(x2, Wqk, bqk.reshape(1, DM), Wv, bv.reshape(1, DM),
                       row_block)

    qk_flat = qk2.reshape(batch, S, H, D).transpose(0, 2, 1, 3).reshape(T, D)
    v_flat = v2.reshape(batch, S, H, D).transpose(0, 2, 1, 3).reshape(T, D)

    rot_flat = rotations.transpose(1, 2, 0, 3).reshape(H, D, n_hashes * C)
    bkt, offs3 = _hash_buckets(qk_flat, rot_flat, n_hashes, C)
    buckets3 = bkt.transpose(1, 0, 2).reshape(n_hashes, 1, T)

    rank3 = _ranks(buckets3, offs3, nb, min(512, T))
    rank_g = rank3.reshape(n_hashes, T)

    kv_flat = jnp.concatenate([qk_flat, v_flat], axis=1)
    kvs = _sc_sort_scatter(kv_flat, rank_g)
    kvs3 = kvs.reshape(n_hashes, T, 2 * D)

    kbf3, vaug3, bks3, mx3 = _prep(kvs3, offs3, min(2048, T))
    bq_block = next(b for b in (768, 512, 256, T) if T % b == 0)
    bk_block = next(b for b in (768, 512, 256, T) if T % b == 0)
    os3 = _attention_sorted(kbf3, vaug3, bks3, offs3, mx3, bq_block, bk_block)

    og = _sc_unsort_gather(os3.reshape(n_hashes * T, 2 * D), rank_g)
    out_flat = _sum_hashes(og.reshape(n_hashes, T, 2 * D), min(1024, T))

    y2 = out_flat.reshape(batch, H, S, D).transpose(0, 2, 1, 3).reshape(
        batch * S, DM)
    out = _outproj(y2, Wo, bo.reshape(1, DM), row_block)
    return out.reshape(batch, S, DM)


# speedup vs baseline: 1.0374x; 1.0374x over previous
import functools
import math

import jax
import jax.numpy as jnp
from jax import lax
from jax.experimental import pallas as pl
from jax.experimental.pallas import tpu as pltpu
from jax.experimental.pallas import tpu_sc as plsc


def _proj_kernel(x_ref, wqk_ref, bqk_ref, wv_ref, bv_ref, qk_ref, v_ref):
    x = x_ref[...]
    qk_ref[...] = jax.lax.dot_general(
        x, wqk_ref[...], (((1,), (1,)), ((), ())),
        preferred_element_type=jnp.float32) + bqk_ref[...]
    v_ref[...] = jax.lax.dot_general(
        x, wv_ref[...], (((1,), (1,)), ((), ())),
        preferred_element_type=jnp.float32) + bv_ref[...]


def _project(x2, Wqk, bqk, Wv, bv, row_block):
    S, DM = x2.shape
    return pl.pallas_call(
        _proj_kernel,
        grid=(S // row_block,),
        in_specs=[
            pl.BlockSpec((row_block, DM), lambda i: (i, 0)),
            pl.BlockSpec((DM, DM), lambda i: (0, 0)),
            pl.BlockSpec((1, DM), lambda i: (0, 0)),
            pl.BlockSpec((DM, DM), lambda i: (0, 0)),
            pl.BlockSpec((1, DM), lambda i: (0, 0)),
        ],
        out_specs=[
            pl.BlockSpec((row_block, DM), lambda i: (i, 0)),
            pl.BlockSpec((row_block, DM), lambda i: (i, 0)),
        ],
        out_shape=[
            jax.ShapeDtypeStruct((S, DM), jnp.float32),
            jax.ShapeDtypeStruct((S, DM), jnp.float32),
        ],
    )(x2, Wqk, bqk, Wv, bv)


def _hash_kernel(qk_ref, rot_ref, bkt_ref, offs_ref, counts_sc,
                 *, n_hashes, rot_size, n_heads, t):
    hh = pl.program_id(0)
    nb = 2 * rot_size
    q = qk_ref[...]
    r = rot_ref[0]
    rot = jax.lax.dot_general(
        q, r, (((1,), (0,)), ((), ())), preferred_element_type=jnp.float32)
    s = rot.shape[0]

    @pl.when(hh == 0)
    def _():
        counts_sc[...] = jnp.zeros_like(counts_sc)

    lanes = lax.broadcasted_iota(jnp.int32, (s, nb), 1)
    for h in range(n_hashes):
        sub = rot[:, h * rot_size:(h + 1) * rot_size]
        full = jnp.concatenate([sub, -sub], axis=1)
        b = jnp.argmax(full, axis=1).astype(jnp.int32)
        bkt_ref[0, h, :] = b
        oh = (b[:, None] == lanes).astype(jnp.float32)
        counts_sc[h, :] = counts_sc[h, :] + jnp.sum(oh, axis=0)

    @pl.when(hh == n_heads - 1)
    def _():
        cnt = counts_sc[...]
        inc = cnt
        shift = 1
        while shift < nb:
            inc = inc + jnp.concatenate(
                [jnp.zeros((n_hashes, shift), jnp.float32),
                 inc[:, :-shift]], axis=1)
            shift *= 2
        offs = jnp.concatenate(
            [jnp.zeros((n_hashes, 1), jnp.float32), inc[:, :-1]], axis=1)
        pad = jnp.full((n_hashes, nb), float(t), dtype=jnp.float32)
        offs_ref[...] = jnp.concatenate(
            [offs, pad], axis=1).astype(jnp.int32)[:, None, :]


def _hash_buckets(qk_heads, rot_flat, n_hashes, rot_size):
    H = rot_flat.shape[0]
    D = rot_flat.shape[1]
    S = qk_heads.shape[0] // H
    nb = 2 * rot_size
    return pl.pallas_call(
        functools.partial(_hash_kernel, n_hashes=n_hashes, rot_size=rot_size,
                          n_heads=H, t=H * S),
        grid=(H,),
        in_specs=[
            pl.BlockSpec((S, D), lambda h: (h, 0)),
            pl.BlockSpec((1, D, n_hashes * rot_size), lambda h: (h, 0, 0)),
        ],
        out_specs=[
            pl.BlockSpec((1, n_hashes, S), lambda h: (h, 0, 0)),
            pl.BlockSpec((n_hashes, 1, 2 * nb), lambda h: (0, 0, 0)),
        ],
        out_shape=[
            jax.ShapeDtypeStruct((H, n_hashes, S), jnp.int32),
            jax.ShapeDtypeStruct((n_hashes, 1, 2 * nb), jnp.int32),
        ],
        scratch_shapes=[pltpu.VMEM((n_hashes, nb), jnp.float32)],
    )(qk_heads, rot_flat)


def _rank_kernel(tri_ref, bkt_ref, offs_ref, rank_ref, counts_sc,
                 *, cs, nb, t, gs):
    h = pl.program_id(0)
    c = pl.program_id(1)

    b = bkt_ref[0, 0, :]
    lanes = lax.broadcasted_iota(jnp.int32, (cs, nb), 1)
    oh = (b[:, None] == lanes).astype(jnp.float32)

    @pl.when(c == 0)
    def _():
        counts_sc[...] = jnp.zeros_like(counts_sc)

    csum = jax.lax.dot_general(
        tri_ref[...], oh.astype(jnp.bfloat16), (((1,), (0,)), ((), ())),
        preferred_element_type=jnp.float32)
    offs = offs_ref[0, 0, :nb].astype(jnp.float32)[None, :]
    inc_global = csum + counts_sc[...]
    rank_f = jnp.sum(oh * (offs + inc_global - 1.0), axis=1)
    rank_ref[0, 0, :] = (rank_f + 0.5).astype(jnp.int32) + (h % gs) * t
    counts_sc[...] = counts_sc[...] + jnp.sum(oh, axis=0, keepdims=True)


def _ranks(buckets3, offs3, nb, cs, gs):
    n_hashes, _, T = buckets3.shape
    nc = T // cs
    rr = lax.broadcasted_iota(jnp.int32, (cs, cs), 0)
    cc = lax.broadcasted_iota(jnp.int32, (cs, cs), 1)
    tri = (rr >= cc).astype(jnp.bfloat16)
    return pl.pallas_call(
        functools.partial(_rank_kernel, cs=cs, nb=nb, t=T, gs=gs),
        grid=(n_hashes, nc),
        in_specs=[
            pl.BlockSpec((cs, cs), lambda h, c: (0, 0)),
            pl.BlockSpec((1, 1, cs), lambda h, c: (h, 0, c)),
            pl.BlockSpec((1, 1, 2 * nb), lambda h, c: (h, 0, 0)),
        ],
        out_specs=pl.BlockSpec((1, 1, cs), lambda h, c: (h, 0, c)),
        out_shape=jax.ShapeDtypeStruct((n_hashes, 1, T), jnp.int32),
        scratch_shapes=[pltpu.VMEM((1, nb), jnp.float32)],
        compiler_params=pltpu.CompilerParams(
            dimension_semantics=("parallel", "arbitrary")),
    )(tri, buckets3, offs3)


def _sc_sort_scatter(kv_flat, rank_g):
    T, DK = kv_flat.shape
    NH = rank_g.shape[0]
    info = plsc.get_sparse_core_info()
    nw = info.num_cores * info.num_subcores
    rpw = T // nw
    nch = rpw // 128
    mesh = plsc.VectorSubcoreMesh(core_axis_name="c", subcore_axis_name="s")

    @functools.partial(
        pl.kernel, mesh=mesh,
        out_type=jax.ShapeDtypeStruct((NH * T, DK), jnp.float32),
        scratch_types=[pltpu.VMEM((nch, 128), jnp.int32),
                       pltpu.VMEM((rpw, DK), jnp.float32),
                       pltpu.SemaphoreType.DMA,
                       pltpu.SemaphoreType.DMA],
    )
    def sortk(kv_hbm, rank_hbm, kvs_hbm, idx_v, rows, semi, semw):
        wid = lax.axis_index("s") * info.num_cores + lax.axis_index("c")
        base = wid * rpw
        pltpu.sync_copy(kv_hbm.at[pl.ds(base, rpw)], rows)

        def per_hash(h, carry):
            loads = [
                pltpu.async_copy(rank_hbm.at[h, pl.ds(base + j * 128, 128)],
                                 idx_v.at[j], semi)
                for j in range(nch)
            ]
            for hd in loads:
                hd.wait()
            stores = [
                pltpu.async_copy(rows.at[pl.ds(j * 128, 128)],
                                 kvs_hbm.at[idx_v.at[j]], semw)
                for j in range(nch)
            ]
            for hd in stores:
                hd.wait()
            return carry

        lax.fori_loop(0, NH, per_hash, 0)

    return sortk(kv_flat, rank_g)


def _sc_unsort_gather(out_sorted, rank_g):
    TT, D = out_sorted.shape
    NH, T = rank_g.shape
    info = plsc.get_sparse_core_info()
    nw = info.num_cores * info.num_subcores
    rpw = T // nw
    nch = rpw // 128
    mesh = plsc.VectorSubcoreMesh(core_axis_name="c", subcore_axis_name="s")

    @functools.partial(
        pl.kernel, mesh=mesh,
        out_type=jax.ShapeDtypeStruct((NH * T, D), jnp.float32),
        scratch_types=[pltpu.VMEM((nch, 128), jnp.int32),
                       pltpu.VMEM((rpw, D), jnp.float32),
                       pltpu.SemaphoreType.DMA,
                       pltpu.SemaphoreType.DMA],
    )
    def gatherk(os_hbm, rank_hbm, og_hbm, idx_v, rows, semi, semr):
        wid = lax.axis_index("s") * info.num_cores + lax.axis_index("c")
        base = wid * rpw

        def per_hash(h, carry):
            loads = [
                pltpu.async_copy(rank_hbm.at[h, pl.ds(base + j * 128, 128)],
                                 idx_v.at[j], semi)
                for j in range(nch)
            ]
            for hd in loads:
                hd.wait()
            reads = [
                pltpu.async_copy(os_hbm.at[idx_v.at[j]],
                                 rows.at[pl.ds(j * 128, 128)], semr)
                for j in range(nch)
            ]
            for hd in reads:
                hd.wait()
            pltpu.sync_copy(rows, og_hbm.at[pl.ds(h * T + base, rpw)])
            return carry

        lax.fori_loop(0, NH, per_hash, 0)

    return gatherk(out_sorted, rank_g)


def _prep_kernel(offs_ref, kv_ref, kbf_ref, vaug_ref, bks_ref, mx_ref,
                 maxn_sc, *, rb, nb, d, nc):
    c = pl.program_id(1)
    kv = kv_ref[0]
    kbf_ref[0] = kv[:, :d].astype(jnp.bfloat16)
    lane = lax.broadcasted_iota(jnp.int32, (rb, 2 * d), 1)
    vaug_ref[0] = jnp.where(
        lane < d, jnp.roll(kv, -d, axis=1),
        jnp.where(lane == d, 1.0, 0.0)).astype(jnp.bfloat16)
    off32 = offs_ref[0, 0, :nb]
    p = c * rb + lax.broadcasted_iota(jnp.int32, (rb, 1), 0)
    bks_ref[0, 0, :] = jnp.sum(
        (off32[None, :] <= p).astype(jnp.int32), axis=1) - 1
    k = kv[:, :d]
    n2 = jnp.max(jnp.sum(k * k, axis=1))

    @pl.when(c == 0)
    def _():
        maxn_sc[...] = jnp.zeros_like(maxn_sc)

    maxn_sc[...] = jnp.maximum(maxn_sc[...], n2.reshape(1, 1))

    @pl.when(c == nc - 1)
    def _():
        mx_ref[...] = jnp.broadcast_to(maxn_sc[...], mx_ref.shape)


def _prep(kvs3, offs3, rb):
    n_hashes, T, DK = kvs3.shape
    d = DK // 2
    nb = offs3.shape[2] // 2
    nc = T // rb
    return pl.pallas_call(
        functools.partial(_prep_kernel, rb=rb, nb=nb, d=d, nc=nc),
        grid=(n_hashes, nc),
        in_specs=[
            pl.BlockSpec((1, 1, 2 * nb), lambda h, c: (h, 0, 0)),
            pl.BlockSpec((1, rb, DK), lambda h, c: (h, c, 0)),
        ],
        out_specs=[
            pl.BlockSpec((1, rb, d), lambda h, c: (h, c, 0)),
            pl.BlockSpec((1, rb, DK), lambda h, c: (h, c, 0)),
            pl.BlockSpec((1, 1, rb), lambda h, c: (h, 0, c)),
            pl.BlockSpec((1, 1, 128), lambda h, c: (h, 0, 0)),
        ],
        out_shape=[
            jax.ShapeDtypeStruct((n_hashes, T, d), jnp.bfloat16),
            jax.ShapeDtypeStruct((n_hashes, T, DK), jnp.bfloat16),
            jax.ShapeDtypeStruct((n_hashes, 1, T), jnp.int32),
            jax.ShapeDtypeStruct((n_hashes, 1, 128), jnp.float32),
        ],
        scratch_shapes=[pltpu.VMEM((1, 1), jnp.float32)],
        compiler_params=pltpu.CompilerParams(
            dimension_semantics=("parallel", "arbitrary")),
    )(offs3, kvs3)


def _attn_kernel(offs_ref, bq_ref, mx_ref, q_ref, k_ref, vaug_ref, bks_ref,
                 o_ref, *, bq, bk, nb, t, d, scale):
    qi = pl.program_id(1)
    off = offs_ref[0, 0, :]
    off32 = off[:nb]
    qlo = qi * bq
    qhi = qlo + bq - 1

    kv_start = jnp.max(jnp.where(off32 <= qlo, off32, 0))
    kv_end = jnp.min(jnp.where(off > qhi, off, t))
    ks_blk = kv_start // bk
    ke_blk = (kv_end + bk - 1) // bk

    bq_id = bq_ref[0, 0, :]
    mshift = mx_ref[0, 0, 0] * scale
    q = q_ref[0] * jnp.bfloat16(scale)

    def body(ki, acc):
        koff = ki * bk
        k = k_ref[0, pl.ds(koff, bk), :]
        vaug = vaug_ref[0, pl.ds(koff, bk), :]
        bk_id = bks_ref[0, 0, pl.ds(koff, bk)]
        s = jax.lax.dot_general(
            q, k, (((1,), (1,)), ((), ())),
            preferred_element_type=jnp.float32)
        mask = bq_id[:, None] == bk_id[None, :]
        p = jnp.exp(jnp.where(mask, s - mshift, -1e30))
        return acc + jax.lax.dot_general(
            p.astype(jnp.bfloat16), vaug, (((1,), (0,)), ((), ())),
            preferred_element_type=jnp.float32)

    a0 = jnp.zeros((bq, 2 * d), dtype=jnp.float32)
    acc = lax.fori_loop(ks_blk, ke_blk, body, a0)
    o_ref[0, :, :d] = acc[:, :d] / acc[:, d:d + 1]


def _attention_sorted(kbf3, vaug3, bks3, offs3, mx3, bq_block, bk_block):
    n_hashes, T, d = kbf3.shape
    DK = 2 * d
    nb = offs3.shape[2] // 2
    nq = T // bq_block
    scale = 1.0 / math.sqrt(d)
    return pl.pallas_call(
        functools.partial(_attn_kernel, bq=bq_block, bk=bk_block,
                          nb=nb, t=T, d=d, scale=scale),
        grid=(n_hashes, nq),
        in_specs=[
            pl.BlockSpec((1, 1, 2 * nb), lambda h, qi: (h, 0, 0)),
            pl.BlockSpec((1, 1, bq_block), lambda h, qi: (h, 0, qi)),
            pl.BlockSpec((1, 1, 128), lambda h, qi: (h, 0, 0)),
            pl.BlockSpec((1, bq_block, d), lambda h, qi: (h, qi, 0)),
            pl.BlockSpec((1, T, d), lambda h, qi: (h, 0, 0)),
            pl.BlockSpec((1, T, DK), lambda h, qi: (h, 0, 0)),
            pl.BlockSpec((1, 1, T), lambda h, qi: (h, 0, 0)),
        ],
        out_specs=pl.BlockSpec((1, bq_block, DK), lambda h, qi: (h, qi, 0)),
        out_shape=jax.ShapeDtypeStruct((n_hashes, T, DK), jnp.float32),
        compiler_params=pltpu.CompilerParams(
            dimension_semantics=("parallel", "parallel")),
    )(offs3, bks3, mx3, kbf3, kbf3, vaug3, bks3)


def _sum_kernel(g_ref, o_ref, *, inv_nh, d):
    o_ref[...] = jnp.sum(g_ref[...][:, :, :d], axis=0) * inv_nh


def _sum_hashes(og_list, n_hashes, row_block):
    NH0, T, DK = og_list[0].shape
    d = DK // 2
    if len(og_list) == 1:
        return pl.pallas_call(
            functools.partial(_sum_kernel, inv_nh=1.0 / n_hashes, d=d),
            grid=(T // row_block,),
            in_specs=[pl.BlockSpec((NH0, row_block, DK), lambda i: (0, i, 0))],
            out_specs=pl.BlockSpec((row_block, d), lambda i: (i, 0)),
            out_shape=jax.ShapeDtypeStruct((T, d), jnp.float32),
        )(og_list[0])
    NH1 = og_list[1].shape[0]

    def body(a_ref, b_ref, o_ref):
        o_ref[...] = (jnp.sum(a_ref[...][:, :, :d], axis=0)
                      + jnp.sum(b_ref[...][:, :, :d], axis=0)) / n_hashes

    return pl.pallas_call(
        body,
        grid=(T // row_block,),
        in_specs=[
            pl.BlockSpec((NH0, row_block, DK), lambda i: (0, i, 0)),
            pl.BlockSpec((NH1, row_block, DK), lambda i: (0, i, 0)),
        ],
        out_specs=pl.BlockSpec((row_block, d), lambda i: (i, 0)),
        out_shape=jax.ShapeDtypeStruct((T, d), jnp.float32),
    )(og_list[0], og_list[1])


def _outproj_kernel(y_ref, wo_ref, bo_ref, o_ref):
    o_ref[...] = jax.lax.dot_general(
        y_ref[...], wo_ref[...], (((1,), (1,)), ((), ())),
        preferred_element_type=jnp.float32) + bo_ref[...]


def _outproj(y2, Wo, bo, row_block):
    S, DM = y2.shape
    return pl.pallas_call(
        _outproj_kernel,
        grid=(S // row_block,),
        in_specs=[
            pl.BlockSpec((row_block, DM), lambda i: (i, 0)),
            pl.BlockSpec((DM, DM), lambda i: (0, 0)),
            pl.BlockSpec((1, DM), lambda i: (0, 0)),
        ],
        out_specs=pl.BlockSpec((row_block, DM), lambda i: (i, 0)),
        out_shape=jax.ShapeDtypeStruct((S, DM), jnp.float32),
    )(y2, Wo, bo)


def kernel(x, Wqk, bqk, Wv, bv, Wo, bo, rotations):
    batch, S, DM = x.shape
    n_hashes, H, D, C = rotations.shape
    T = batch * H * S
    nb = 2 * C

    x2 = x.reshape(batch * S, DM)
    row_block = min(256, batch * S)
    qk2, v2 = _project(x2, Wqk, bqk.reshape(1, DM), Wv, bv.reshape(1, DM),
                       row_block)

    qk_flat = qk2.reshape(batch, S, H, D).transpose(0, 2, 1, 3).reshape(T, D)
    v_flat = v2.reshape(batch, S, H, D).transpose(0, 2, 1, 3).reshape(T, D)

    rot_flat = rotations.transpose(1, 2, 0, 3).reshape(H, D, n_hashes * C)
    bkt, offs3 = _hash_buckets(qk_flat, rot_flat, n_hashes, C)
    buckets3 = bkt.transpose(1, 0, 2).reshape(n_hashes, 1, T)

    gs = n_hashes // 2 if n_hashes % 2 == 0 else n_hashes
    rank3 = _ranks(buckets3, offs3, nb, min(512, T), gs)
    rank_g = rank3.reshape(n_hashes, T)

    kv_flat = jnp.concatenate([qk_flat, v_flat], axis=1)
    bq_block = next(b for b in (768, 512, 256, T) if T % b == 0)
    bk_block = next(b for b in (768, 512, 256, T) if T % b == 0)
    groups = [(0, gs)] + ([(gs, n_hashes)] if gs < n_hashes else [])
    og_list = []
    for a, b in groups:
        kvs = _sc_sort_scatter(kv_flat, rank_g[a:b])
        kvs3 = kvs.reshape(b - a, T, 2 * D)
        kbf3, vaug3, bks3, mx3 = _prep(kvs3, offs3[a:b], min(2048, T))
        os3 = _attention_sorted(kbf3, vaug3, bks3, offs3[a:b], mx3,
                                bq_block, bk_block)
        og = _sc_unsort_gather(os3.reshape((b - a) * T, 2 * D), rank_g[a:b])
        og_list.append(og.reshape(b - a, T, 2 * D))
    out_flat = _sum_hashes(og_list, n_hashes, min(1024, T))

    y2 = out_flat.reshape(batch, H, S, D).transpose(0, 2, 1, 3).reshape(
        batch * S, DM)
    out = _outproj(y2, Wo, bo.reshape(1, DM), row_block)
    return out.reshape(batch, S, DM)


# speedup vs baseline: 1.0423x; 1.0047x over previous
import functools
import math

import jax
import jax.numpy as jnp
from jax import lax
from jax.experimental import pallas as pl
from jax.experimental.pallas import tpu as pltpu
from jax.experimental.pallas import tpu_sc as plsc


def _proj_kernel(x_ref, wqk_ref, bqk_ref, wv_ref, bv_ref, qk_ref, v_ref):
    x = x_ref[...]
    qk_ref[...] = jax.lax.dot_general(
        x, wqk_ref[...], (((1,), (1,)), ((), ())),
        preferred_element_type=jnp.float32) + bqk_ref[...]
    v_ref[...] = jax.lax.dot_general(
        x, wv_ref[...], (((1,), (1,)), ((), ())),
        preferred_element_type=jnp.float32) + bv_ref[...]


def _project(x2, Wqk, bqk, Wv, bv, row_block):
    S, DM = x2.shape
    return pl.pallas_call(
        _proj_kernel,
        grid=(S // row_block,),
        in_specs=[
            pl.BlockSpec((row_block, DM), lambda i: (i, 0)),
            pl.BlockSpec((DM, DM), lambda i: (0, 0)),
            pl.BlockSpec((1, DM), lambda i: (0, 0)),
            pl.BlockSpec((DM, DM), lambda i: (0, 0)),
            pl.BlockSpec((1, DM), lambda i: (0, 0)),
        ],
        out_specs=[
            pl.BlockSpec((row_block, DM), lambda i: (i, 0)),
            pl.BlockSpec((row_block, DM), lambda i: (i, 0)),
        ],
        out_shape=[
            jax.ShapeDtypeStruct((S, DM), jnp.float32),
            jax.ShapeDtypeStruct((S, DM), jnp.float32),
        ],
    )(x2, Wqk, bqk, Wv, bv)


def _hash_kernel(qk_ref, rot_ref, bkt_ref, offs_ref, counts_sc,
                 *, n_hashes, rot_size, n_heads, t):
    hh = pl.program_id(0)
    nb = 2 * rot_size
    q = qk_ref[...]
    r = rot_ref[0]
    rot = jax.lax.dot_general(
        q, r, (((1,), (0,)), ((), ())), preferred_element_type=jnp.float32)
    s = rot.shape[0]

    @pl.when(hh == 0)
    def _():
        counts_sc[...] = jnp.zeros_like(counts_sc)

    lanes = lax.broadcasted_iota(jnp.int32, (s, nb), 1)
    for h in range(n_hashes):
        sub = rot[:, h * rot_size:(h + 1) * rot_size]
        full = jnp.concatenate([sub, -sub], axis=1)
        b = jnp.argmax(full, axis=1).astype(jnp.int32)
        bkt_ref[0, h, :] = b
        oh = (b[:, None] == lanes).astype(jnp.float32)
        counts_sc[h, :] = counts_sc[h, :] + jnp.sum(oh, axis=0)

    @pl.when(hh == n_heads - 1)
    def _():
        cnt = counts_sc[...]
        inc = cnt
        shift = 1
        while shift < nb:
            inc = inc + jnp.concatenate(
                [jnp.zeros((n_hashes, shift), jnp.float32),
                 inc[:, :-shift]], axis=1)
            shift *= 2
        offs = jnp.concatenate(
            [jnp.zeros((n_hashes, 1), jnp.float32), inc[:, :-1]], axis=1)
        pad = jnp.full((n_hashes, nb), float(t), dtype=jnp.float32)
        offs_ref[...] = jnp.concatenate(
            [offs, pad], axis=1).astype(jnp.int32)[:, None, :]


def _hash_buckets(qk_heads, rot_flat, n_hashes, rot_size):
    H = rot_flat.shape[0]
    D = rot_flat.shape[1]
    S = qk_heads.shape[0] // H
    nb = 2 * rot_size
    return pl.pallas_call(
        functools.partial(_hash_kernel, n_hashes=n_hashes, rot_size=rot_size,
                          n_heads=H, t=H * S),
        grid=(H,),
        in_specs=[
            pl.BlockSpec((S, D), lambda h: (h, 0)),
            pl.BlockSpec((1, D, n_hashes * rot_size), lambda h: (h, 0, 0)),
        ],
        out_specs=[
            pl.BlockSpec((1, n_hashes, S), lambda h: (h, 0, 0)),
            pl.BlockSpec((n_hashes, 1, 2 * nb), lambda h: (0, 0, 0)),
        ],
        out_shape=[
            jax.ShapeDtypeStruct((H, n_hashes, S), jnp.int32),
            jax.ShapeDtypeStruct((n_hashes, 1, 2 * nb), jnp.int32),
        ],
        scratch_shapes=[pltpu.VMEM((n_hashes, nb), jnp.float32)],
    )(qk_heads, rot_flat)


def _rank_kernel(tri_ref, bkt_ref, offs_ref, rank_ref, counts_sc,
                 *, cs, nb, t, gs):
    h = pl.program_id(0)
    c = pl.program_id(1)

    b = bkt_ref[0, 0, :]
    lanes = lax.broadcasted_iota(jnp.int32, (cs, nb), 1)
    oh = (b[:, None] == lanes).astype(jnp.float32)

    @pl.when(c == 0)
    def _():
        counts_sc[...] = jnp.zeros_like(counts_sc)

    csum = jax.lax.dot_general(
        tri_ref[...], oh.astype(jnp.bfloat16), (((1,), (0,)), ((), ())),
        preferred_element_type=jnp.float32)
    offs = offs_ref[0, 0, :nb].astype(jnp.float32)[None, :]
    inc_global = csum + counts_sc[...]
    rank_f = jnp.sum(oh * (offs + inc_global - 1.0), axis=1)
    rank_ref[0, 0, :] = (rank_f + 0.5).astype(jnp.int32) + (h % gs) * t
    counts_sc[...] = counts_sc[...] + jnp.sum(oh, axis=0, keepdims=True)


def _ranks(buckets3, offs3, nb, cs, gs):
    n_hashes, _, T = buckets3.shape
    nc = T // cs
    rr = lax.broadcasted_iota(jnp.int32, (cs, cs), 0)
    cc = lax.broadcasted_iota(jnp.int32, (cs, cs), 1)
    tri = (rr >= cc).astype(jnp.bfloat16)
    return pl.pallas_call(
        functools.partial(_rank_kernel, cs=cs, nb=nb, t=T, gs=gs),
        grid=(n_hashes, nc),
        in_specs=[
            pl.BlockSpec((cs, cs), lambda h, c: (0, 0)),
            pl.BlockSpec((1, 1, cs), lambda h, c: (h, 0, c)),
            pl.BlockSpec((1, 1, 2 * nb), lambda h, c: (h, 0, 0)),
        ],
        out_specs=pl.BlockSpec((1, 1, cs), lambda h, c: (h, 0, c)),
        out_shape=jax.ShapeDtypeStruct((n_hashes, 1, T), jnp.int32),
        scratch_shapes=[pltpu.VMEM((1, nb), jnp.float32)],
        compiler_params=pltpu.CompilerParams(
            dimension_semantics=("parallel", "arbitrary")),
    )(tri, buckets3, offs3)


def _sc_sort_scatter(kv_flat, rank_g):
    T, DK = kv_flat.shape
    NH = rank_g.shape[0]
    info = plsc.get_sparse_core_info()
    nw = info.num_cores * info.num_subcores
    rpw = T // nw
    nch = rpw // 128
    mesh = plsc.VectorSubcoreMesh(core_axis_name="c", subcore_axis_name="s")

    @functools.partial(
        pl.kernel, mesh=mesh,
        out_type=jax.ShapeDtypeStruct((NH * T, DK), jnp.float32),
        scratch_types=[pltpu.VMEM((nch, 128), jnp.int32),
                       pltpu.VMEM((rpw, DK), jnp.float32),
                       pltpu.SemaphoreType.DMA,
                       pltpu.SemaphoreType.DMA],
    )
    def sortk(kv_hbm, rank_hbm, kvs_hbm, idx_v, rows, semi, semw):
        wid = lax.axis_index("s") * info.num_cores + lax.axis_index("c")
        base = wid * rpw
        pltpu.sync_copy(kv_hbm.at[pl.ds(base, rpw)], rows)

        def per_hash(h, carry):
            loads = [
                pltpu.async_copy(rank_hbm.at[h, pl.ds(base + j * 128, 128)],
                                 idx_v.at[j], semi)
                for j in range(nch)
            ]
            for hd in loads:
                hd.wait()
            stores = [
                pltpu.async_copy(rows.at[pl.ds(j * 128, 128)],
                                 kvs_hbm.at[idx_v.at[j]], semw)
                for j in range(nch)
            ]
            for hd in stores:
                hd.wait()
            return carry

        lax.fori_loop(0, NH, per_hash, 0)

    return sortk(kv_flat, rank_g)


def _sc_unsort_gather(out_sorted, rank_g):
    TT, D = out_sorted.shape
    NH, T = rank_g.shape
    info = plsc.get_sparse_core_info()
    nw = info.num_cores * info.num_subcores
    rpw = T // nw
    nch = rpw // 128
    mesh = plsc.VectorSubcoreMesh(core_axis_name="c", subcore_axis_name="s")

    @functools.partial(
        pl.kernel, mesh=mesh,
        out_type=jax.ShapeDtypeStruct((NH * T, D), jnp.float32),
        scratch_types=[pltpu.VMEM((nch, 128), jnp.int32),
                       pltpu.VMEM((rpw, D), jnp.float32),
                       pltpu.SemaphoreType.DMA,
                       pltpu.SemaphoreType.DMA],
    )
    def gatherk(os_hbm, rank_hbm, og_hbm, idx_v, rows, semi, semr):
        wid = lax.axis_index("s") * info.num_cores + lax.axis_index("c")
        base = wid * rpw

        def per_hash(h, carry):
            loads = [
                pltpu.async_copy(rank_hbm.at[h, pl.ds(base + j * 128, 128)],
                                 idx_v.at[j], semi)
                for j in range(nch)
            ]
            for hd in loads:
                hd.wait()
            reads = [
                pltpu.async_copy(os_hbm.at[idx_v.at[j]],
                                 rows.at[pl.ds(j * 128, 128)], semr)
                for j in range(nch)
            ]
            for hd in reads:
                hd.wait()
            pltpu.sync_copy(rows, og_hbm.at[pl.ds(h * T + base, rpw)])
            return carry

        lax.fori_loop(0, NH, per_hash, 0)

    return gatherk(out_sorted, rank_g)


def _prep_kernel(offs_ref, kv_ref, kbf_ref, vaug_ref, bks_ref, mx_ref,
                 maxn_sc, *, rb, nb, d, nc):
    c = pl.program_id(1)
    kv = kv_ref[0]
    kbf_ref[0] = kv[:, :d].astype(jnp.bfloat16)
    lane = lax.broadcasted_iota(jnp.int32, (rb, 2 * d), 1)
    vaug_ref[0] = jnp.where(
        lane < d, jnp.roll(kv, -d, axis=1),
        jnp.where(lane == d, 1.0, 0.0)).astype(jnp.bfloat16)
    off32 = offs_ref[0, 0, :nb]
    p = c * rb + lax.broadcasted_iota(jnp.int32, (rb, 1), 0)
    bks_ref[0, 0, :] = jnp.sum(
        (off32[None, :] <= p).astype(jnp.int32), axis=1) - 1
    k = kv[:, :d]
    n2 = jnp.max(jnp.sum(k * k, axis=1))

    @pl.when(c == 0)
    def _():
        maxn_sc[...] = jnp.zeros_like(maxn_sc)

    maxn_sc[...] = jnp.maximum(maxn_sc[...], n2.reshape(1, 1))

    @pl.when(c == nc - 1)
    def _():
        mx_ref[...] = jnp.broadcast_to(maxn_sc[...], mx_ref.shape)


def _prep(kvs3, offs3, rb):
    n_hashes, T, DK = kvs3.shape
    d = DK // 2
    nb = offs3.shape[2] // 2
    nc = T // rb
    return pl.pallas_call(
        functools.partial(_prep_kernel, rb=rb, nb=nb, d=d, nc=nc),
        grid=(n_hashes, nc),
        in_specs=[
            pl.BlockSpec((1, 1, 2 * nb), lambda h, c: (h, 0, 0)),
            pl.BlockSpec((1, rb, DK), lambda h, c: (h, c, 0)),
        ],
        out_specs=[
            pl.BlockSpec((1, rb, d), lambda h, c: (h, c, 0)),
            pl.BlockSpec((1, rb, DK), lambda h, c: (h, c, 0)),
            pl.BlockSpec((1, 1, rb), lambda h, c: (h, 0, c)),
            pl.BlockSpec((1, 1, 128), lambda h, c: (h, 0, 0)),
        ],
        out_shape=[
            jax.ShapeDtypeStruct((n_hashes, T, d), jnp.bfloat16),
            jax.ShapeDtypeStruct((n_hashes, T, DK), jnp.bfloat16),
            jax.ShapeDtypeStruct((n_hashes, 1, T), jnp.int32),
            jax.ShapeDtypeStruct((n_hashes, 1, 128), jnp.float32),
        ],
        scratch_shapes=[pltpu.VMEM((1, 1), jnp.float32)],
        compiler_params=pltpu.CompilerParams(
            dimension_semantics=("parallel", "arbitrary")),
    )(offs3, kvs3)


def _attn_kernel(offs_ref, bq_ref, mx_ref, q_ref, k_ref, vaug_ref, bks_ref,
                 o_ref, *, bq, bk, nb, t, d, scale):
    qi = pl.program_id(1)
    off = offs_ref[0, 0, :]
    off32 = off[:nb]
    qlo = qi * bq
    qhi = qlo + bq - 1

    kv_start = jnp.max(jnp.where(off32 <= qlo, off32, 0))
    kv_end = jnp.min(jnp.where(off > qhi, off, t))
    ks_blk = kv_start // bk
    ke_blk = (kv_end + bk - 1) // bk

    bq_id = bq_ref[0, 0, :]
    mshift = mx_ref[0, 0, 0] * scale
    q = q_ref[0] * jnp.bfloat16(scale)

    def body(ki, acc):
        koff = ki * bk
        k = k_ref[0, pl.ds(koff, bk), :]
        vaug = vaug_ref[0, pl.ds(koff, bk), :]
        bk_id = bks_ref[0, 0, pl.ds(koff, bk)]
        s = jax.lax.dot_general(
            q, k, (((1,), (1,)), ((), ())),
            preferred_element_type=jnp.float32)
        mask = bq_id[:, None] == bk_id[None, :]
        p = jnp.exp(jnp.where(mask, s - mshift, -1e30))
        return acc + jax.lax.dot_general(
            p.astype(jnp.bfloat16), vaug, (((1,), (0,)), ((), ())),
            preferred_element_type=jnp.float32)

    a0 = jnp.zeros((bq, 2 * d), dtype=jnp.float32)
    acc = lax.fori_loop(ks_blk, ke_blk, body, a0)
    o_ref[0, :, :d] = acc[:, :d] / acc[:, d:d + 1]


def _attention_sorted(kbf3, vaug3, bks3, offs3, mx3, bq_block, bk_block):
    n_hashes, T, d = kbf3.shape
    DK = 2 * d
    nb = offs3.shape[2] // 2
    nq = T // bq_block
    scale = 1.0 / math.sqrt(d)
    return pl.pallas_call(
        functools.partial(_attn_kernel, bq=bq_block, bk=bk_block,
                          nb=nb, t=T, d=d, scale=scale),
        grid=(n_hashes, nq),
        in_specs=[
            pl.BlockSpec((1, 1, 2 * nb), lambda h, qi: (h, 0, 0)),
            pl.BlockSpec((1, 1, bq_block), lambda h, qi: (h, 0, qi)),
            pl.BlockSpec((1, 1, 128), lambda h, qi: (h, 0, 0)),
            pl.BlockSpec((1, bq_block, d), lambda h, qi: (h, qi, 0)),
            pl.BlockSpec((1, T, d), lambda h, qi: (h, 0, 0)),
            pl.BlockSpec((1, T, DK), lambda h, qi: (h, 0, 0)),
            pl.BlockSpec((1, 1, T), lambda h, qi: (h, 0, 0)),
        ],
        out_specs=pl.BlockSpec((1, bq_block, DK), lambda h, qi: (h, qi, 0)),
        out_shape=jax.ShapeDtypeStruct((n_hashes, T, DK), jnp.float32),
        compiler_params=pltpu.CompilerParams(
            dimension_semantics=("parallel", "parallel")),
    )(offs3, bks3, mx3, kbf3, kbf3, vaug3, bks3)


def _sum_kernel(g_ref, o_ref, *, inv_nh, d):
    o_ref[...] = jnp.sum(g_ref[...][:, :, :d], axis=0) * inv_nh


def _sum_hashes(og_list, n_hashes, row_block):
    T, DK = og_list[0].shape[1], og_list[0].shape[2]
    d = DK // 2

    def body(*refs):
        o_ref = refs[-1]
        acc = jnp.sum(refs[0][...][:, :, :d], axis=0)
        for r in refs[1:-1]:
            acc = acc + jnp.sum(r[...][:, :, :d], axis=0)
        o_ref[...] = acc / n_hashes

    return pl.pallas_call(
        body,
        grid=(T // row_block,),
        in_specs=[pl.BlockSpec((g.shape[0], row_block, DK),
                               lambda i: (0, i, 0)) for g in og_list],
        out_specs=pl.BlockSpec((row_block, d), lambda i: (i, 0)),
        out_shape=jax.ShapeDtypeStruct((T, d), jnp.float32),
    )(*og_list)


def _outproj_kernel(y_ref, wo_ref, bo_ref, o_ref):
    o_ref[...] = jax.lax.dot_general(
        y_ref[...], wo_ref[...], (((1,), (1,)), ((), ())),
        preferred_element_type=jnp.float32) + bo_ref[...]


def _outproj(y2, Wo, bo, row_block):
    S, DM = y2.shape
    return pl.pallas_call(
        _outproj_kernel,
        grid=(S // row_block,),
        in_specs=[
            pl.BlockSpec((row_block, DM), lambda i: (i, 0)),
            pl.BlockSpec((DM, DM), lambda i: (0, 0)),
            pl.BlockSpec((1, DM), lambda i: (0, 0)),
        ],
        out_specs=pl.BlockSpec((row_block, DM), lambda i: (i, 0)),
        out_shape=jax.ShapeDtypeStruct((S, DM), jnp.float32),
    )(y2, Wo, bo)


def kernel(x, Wqk, bqk, Wv, bv, Wo, bo, rotations):
    batch, S, DM = x.shape
    n_hashes, H, D, C = rotations.shape
    T = batch * H * S
    nb = 2 * C

    x2 = x.reshape(batch * S, DM)
    row_block = min(256, batch * S)
    qk2, v2 = _project(x2, Wqk, bqk.reshape(1, DM), Wv, bv.reshape(1, DM),
                       row_block)

    qk_flat = qk2.reshape(batch, S, H, D).transpose(0, 2, 1, 3).reshape(T, D)
    v_flat = v2.reshape(batch, S, H, D).transpose(0, 2, 1, 3).reshape(T, D)

    rot_flat = rotations.transpose(1, 2, 0, 3).reshape(H, D, n_hashes * C)
    bkt, offs3 = _hash_buckets(qk_flat, rot_flat, n_hashes, C)
    buckets3 = bkt.transpose(1, 0, 2).reshape(n_hashes, 1, T)

    gs = 2 if n_hashes % 2 == 0 else n_hashes
    rank3 = _ranks(buckets3, offs3, nb, min(512, T), gs)
    rank_g = rank3.reshape(n_hashes, T)

    kv_flat = jnp.concatenate([qk_flat, v_flat], axis=1)
    bq_block = next(b for b in (768, 512, 256, T) if T % b == 0)
    bk_block = next(b for b in (768, 512, 256, T) if T % b == 0)
    groups = [(a, min(a + gs, n_hashes)) for a in range(0, n_hashes, gs)]
    og_list = []
    for a, b in groups:
        kvs = _sc_sort_scatter(kv_flat, rank_g[a:b])
        kvs3 = kvs.reshape(b - a, T, 2 * D)
        kbf3, vaug3, bks3, mx3 = _prep(kvs3, offs3[a:b], min(2048, T))
        os3 = _attention_sorted(kbf3, vaug3, bks3, offs3[a:b], mx3,
                                bq_block, bk_block)
        og = _sc_unsort_gather(os3.reshape((b - a) * T, 2 * D), rank_g[a:b])
        og_list.append(og.reshape(b - a, T, 2 * D))
    out_flat = _sum_hashes(og_list, n_hashes, min(1024, T))

    y2 = out_flat.reshape(batch, H, S, D).transpose(0, 2, 1, 3).reshape(
        batch * S, DM)
    out = _outproj(y2, Wo, bo.reshape(1, DM), row_block)
    return out.reshape(batch, S, DM)
